# Initial kernel scaffold; baseline (speedup 1.0000x reference)
import jax
import jax.numpy as jnp
from jax.experimental import pallas as pl


def kernel(x, W, b):
    raise NotImplementedError("write your pallas kernel here")


# speedup vs baseline: 1.0823x; 1.0823x over previous
import numpy as np
import jax
import jax.numpy as jnp
from jax.experimental import pallas as pl

H = 384
W_IMG = 384
C = 3
P = 16
EMBED = 192
N_TARGETS = 4
HP = H // P
WP = W_IMG // P
NPATCH = HP * WP
PATCH_DIM = P * P * C


def _rect_indices(rng, h, w, sfr, arr):
    low_w = int(w * sfr[0])
    high_w = int(w * sfr[1])
    rec_width = int(rng.integers(max(low_w, 1), high_w + 1))
    low_h = int(rec_width * arr[0])
    high_h = int(rec_width * arr[1])
    rec_height = int(rng.integers(max(low_h, 1), high_h + 1))
    rec_height = min(rec_height, h)
    start_w = int(rng.integers(0, w - rec_width + 1))
    start_h = int(rng.integers(0, h - rec_height + 1))
    start = start_h * w + start_w
    idx = np.concatenate(
        [np.arange(start + i * w, start + i * w + rec_width) for i in range(rec_height)]
    )
    return idx.astype(np.int64)


def _mask_indices():
    rng = np.random.default_rng(0)
    targets = [
        _rect_indices(rng, HP, WP, (0.15, 0.2), (0.75, 1.5)) for _ in range(N_TARGETS)
    ]
    ctx = _rect_indices(rng, HP, WP, (0.85, 1.0), (1.0, 1.0))
    all_t = np.concatenate(targets)
    ctx = ctx[~np.isin(ctx, all_t)]
    return ctx, np.concatenate(targets)


CTX_IDX, TGT_IDX = _mask_indices()
N_CTX = len(CTX_IDX)
N_TGT = len(TGT_IDX)


def _runs(idx):
    runs = []
    start = 0
    for i in range(1, len(idx) + 1):
        if i == len(idx) or idx[i] != idx[i - 1] + 1:
            runs.append((int(idx[start]), start, i - start))
            start = i
    return runs


CTX_RUNS = _runs(CTX_IDX)
TGT_RUNS = _runs(TGT_IDX)


def _proj_gather_kernel(p_ref, w_ref, b_ref, ctx_ref, tgt_ref):
    emb = (
        jnp.dot(p_ref[0], w_ref[...], preferred_element_type=jnp.float32)
        + b_ref[...]
    )
    for src, dst, ln in CTX_RUNS:
        ctx_ref[0, dst : dst + ln, :] = emb[src : src + ln, :]
    for src, dst, ln in TGT_RUNS:
        tgt_ref[0, dst : dst + ln, :] = emb[src : src + ln, :]


def kernel(x, W, b):
    B = x.shape[0]
    xp = x.reshape(B, C, HP, P, WP, P)
    xp = jnp.transpose(xp, (0, 2, 4, 3, 5, 1))
    patches = xp.reshape(B, NPATCH, PATCH_DIM)
    b2 = b.reshape(1, EMBED)

    grid = (B,)
    out = pl.pallas_call(
        _proj_gather_kernel,
        grid=grid,
        in_specs=[
            pl.BlockSpec((1, NPATCH, PATCH_DIM), lambda n: (n, 0, 0)),
            pl.BlockSpec((PATCH_DIM, EMBED), lambda n: (0, 0)),
            pl.BlockSpec((1, EMBED), lambda n: (0, 0)),
        ],
        out_specs=[
            pl.BlockSpec((1, N_CTX, EMBED), lambda n: (n, 0, 0)),
            pl.BlockSpec((1, N_TGT, EMBED), lambda n: (n, 0, 0)),
        ],
        out_shape=[
            jax.ShapeDtypeStruct((B, N_CTX, EMBED), jnp.float32),
            jax.ShapeDtypeStruct((B, N_TGT, EMBED), jnp.float32),
        ],
    )(patches, W, b2)
    return (out[0], out[1])


# speedup vs baseline: 1.1447x; 1.0577x over previous
import numpy as np
import jax
import jax.numpy as jnp
from jax.experimental import pallas as pl

H = 384
W_IMG = 384
C = 3
P = 16
EMBED = 192
N_TARGETS = 4
HP = H // P
WP = W_IMG // P
NPATCH = HP * WP
PATCH_DIM = P * P * C


def _rect_indices(rng, h, w, sfr, arr):
    low_w = int(w * sfr[0])
    high_w = int(w * sfr[1])
    rec_width = int(rng.integers(max(low_w, 1), high_w + 1))
    low_h = int(rec_width * arr[0])
    high_h = int(rec_width * arr[1])
    rec_height = int(rng.integers(max(low_h, 1), high_h + 1))
    rec_height = min(rec_height, h)
    start_w = int(rng.integers(0, w - rec_width + 1))
    start_h = int(rng.integers(0, h - rec_height + 1))
    start = start_h * w + start_w
    idx = np.concatenate(
        [np.arange(start + i * w, start + i * w + rec_width) for i in range(rec_height)]
    )
    return idx.astype(np.int64)


def _mask_indices():
    rng = np.random.default_rng(0)
    targets = [
        _rect_indices(rng, HP, WP, (0.15, 0.2), (0.75, 1.5)) for _ in range(N_TARGETS)
    ]
    ctx = _rect_indices(rng, HP, WP, (0.85, 1.0), (1.0, 1.0))
    all_t = np.concatenate(targets)
    ctx = ctx[~np.isin(ctx, all_t)]
    return ctx, np.concatenate(targets)


CTX_IDX, TGT_IDX = _mask_indices()
N_CTX = len(CTX_IDX)
N_TGT = len(TGT_IDX)


def _runs(idx):
    runs = []
    start = 0
    for i in range(1, len(idx) + 1):
        if i == len(idx) or idx[i] != idx[i - 1] + 1:
            runs.append((int(idx[start]), start, i - start))
            start = i
    return runs


CTX_RUNS = _runs(CTX_IDX)
TGT_RUNS = _runs(TGT_IDX)


BN = 4


def _proj_gather_kernel(p_ref, w_ref, b_ref, ctx_ref, tgt_ref):
    for i in range(BN):
        emb = (
            jnp.dot(p_ref[i], w_ref[...], preferred_element_type=jnp.float32)
            + b_ref[...]
        )
        for src, dst, ln in CTX_RUNS:
            ctx_ref[i, dst : dst + ln, :] = emb[src : src + ln, :]
        for src, dst, ln in TGT_RUNS:
            tgt_ref[i, dst : dst + ln, :] = emb[src : src + ln, :]


def kernel(x, W, b):
    B = x.shape[0]
    xp = x.reshape(B, C, HP, P, WP, P)
    xp = jnp.transpose(xp, (0, 2, 4, 3, 5, 1))
    patches = xp.reshape(B, NPATCH, PATCH_DIM)
    b2 = b.reshape(1, EMBED)

    grid = (B // BN,)
    out = pl.pallas_call(
        _proj_gather_kernel,
        grid=grid,
        in_specs=[
            pl.BlockSpec((BN, NPATCH, PATCH_DIM), lambda n: (n, 0, 0)),
            pl.BlockSpec((PATCH_DIM, EMBED), lambda n: (0, 0)),
            pl.BlockSpec((1, EMBED), lambda n: (0, 0)),
        ],
        out_specs=[
            pl.BlockSpec((BN, N_CTX, EMBED), lambda n: (n, 0, 0)),
            pl.BlockSpec((BN, N_TGT, EMBED), lambda n: (n, 0, 0)),
        ],
        out_shape=[
            jax.ShapeDtypeStruct((B, N_CTX, EMBED), jnp.float32),
            jax.ShapeDtypeStruct((B, N_TGT, EMBED), jnp.float32),
        ],
    )(patches, W, b2)
    return (out[0], out[1])


# speedup vs baseline: 1.3347x; 1.1660x over previous
import numpy as np
import jax
import jax.numpy as jnp
from jax.experimental import pallas as pl

H = 384
W_IMG = 384
C = 3
P = 16
EMBED = 192
N_TARGETS = 4
HP = H // P
WP = W_IMG // P
NPATCH = HP * WP
PATCH_DIM = P * P * C


def _rect_indices(rng, h, w, sfr, arr):
    low_w = int(w * sfr[0])
    high_w = int(w * sfr[1])
    rec_width = int(rng.integers(max(low_w, 1), high_w + 1))
    low_h = int(rec_width * arr[0])
    high_h = int(rec_width * arr[1])
    rec_height = int(rng.integers(max(low_h, 1), high_h + 1))
    rec_height = min(rec_height, h)
    start_w = int(rng.integers(0, w - rec_width + 1))
    start_h = int(rng.integers(0, h - rec_height + 1))
    start = start_h * w + start_w
    idx = np.concatenate(
        [np.arange(start + i * w, start + i * w + rec_width) for i in range(rec_height)]
    )
    return idx.astype(np.int64)


def _mask_indices():
    rng = np.random.default_rng(0)
    targets = [
        _rect_indices(rng, HP, WP, (0.15, 0.2), (0.75, 1.5)) for _ in range(N_TARGETS)
    ]
    ctx = _rect_indices(rng, HP, WP, (0.85, 1.0), (1.0, 1.0))
    all_t = np.concatenate(targets)
    ctx = ctx[~np.isin(ctx, all_t)]
    return ctx, np.concatenate(targets)


CTX_IDX, TGT_IDX = _mask_indices()
N_CTX = len(CTX_IDX)
N_TGT = len(TGT_IDX)


def _runs(idx):
    runs = []
    start = 0
    for i in range(1, len(idx) + 1):
        if i == len(idx) or idx[i] != idx[i - 1] + 1:
            runs.append((int(idx[start]), start, i - start))
            start = i
    return runs


CTX_RUNS = _runs(CTX_IDX)
TGT_RUNS = _runs(TGT_IDX)


BN = 4


def _proj_gather_kernel(p_ref, w_ref, b_ref, ctx_ref, tgt_ref):
    for i in range(BN):
        emb = (
            jnp.dot(p_ref[i], w_ref[...], preferred_element_type=jnp.float32)
            + b_ref[...]
        )
        for src, dst, ln in CTX_RUNS:
            ctx_ref[i, dst : dst + ln, :] = emb[src : src + ln, :]
        for src, dst, ln in TGT_RUNS:
            tgt_ref[i, dst : dst + ln, :] = emb[src : src + ln, :]


def kernel(x, W, b):
    B = x.shape[0]
    xp = x.reshape(B, C, HP, P, WP, P)
    xp = jnp.transpose(xp, (0, 2, 4, 3, 5, 1))
    patches = xp.reshape(B, NPATCH, PATCH_DIM).astype(jnp.bfloat16)
    b2 = b.reshape(1, EMBED)
    Wb = W.astype(jnp.bfloat16)

    grid = (B // BN,)
    out = pl.pallas_call(
        _proj_gather_kernel,
        grid=grid,
        in_specs=[
            pl.BlockSpec((BN, NPATCH, PATCH_DIM), lambda n: (n, 0, 0)),
            pl.BlockSpec((PATCH_DIM, EMBED), lambda n: (0, 0)),
            pl.BlockSpec((1, EMBED), lambda n: (0, 0)),
        ],
        out_specs=[
            pl.BlockSpec((BN, N_CTX, EMBED), lambda n: (n, 0, 0)),
            pl.BlockSpec((BN, N_TGT, EMBED), lambda n: (n, 0, 0)),
        ],
        out_shape=[
            jax.ShapeDtypeStruct((B, N_CTX, EMBED), jnp.float32),
            jax.ShapeDtypeStruct((B, N_TGT, EMBED), jnp.float32),
        ],
    )(patches, Wb, b2)
    return (out[0], out[1])
